# Initial kernel scaffold; baseline (speedup 1.0000x reference)
import jax
import jax.numpy as jnp
from jax.experimental import pallas as pl


def kernel(indices, weight):
    raise NotImplementedError("write your pallas kernel here")


# speedup vs baseline: 6.3951x; 6.3951x over previous
import functools

import jax
import jax.numpy as jnp
from jax import lax
from jax.experimental import pallas as pl
from jax.experimental.pallas import tpu as pltpu
from jax.experimental.pallas import tpu_sc as plsc

_NUM_CORES = 2
_NUM_SUBCORES = 16
_NW = _NUM_CORES * _NUM_SUBCORES

_CHUNK = 512


def _make_gather(num_rows, dim, batch):
    assert batch % (_NW * _CHUNK) == 0
    b_per_w = batch // _NW
    nchunks = b_per_w // _CHUNK
    mesh = plsc.VectorSubcoreMesh(core_axis_name="c", subcore_axis_name="s")

    @functools.partial(
        pl.kernel,
        mesh=mesh,
        compiler_params=pltpu.CompilerParams(use_tc_tiling_on_sc=False),
        out_type=jax.ShapeDtypeStruct((batch, dim), jnp.float32),
        scratch_types=[
            pltpu.VMEM((b_per_w,), jnp.int32),
            pltpu.VMEM((2, _CHUNK, dim), jnp.float32),
            pltpu.SemaphoreType.DMA,
            pltpu.SemaphoreType.DMA,
        ],
    )
    def gather(table_hbm, idx_hbm, out_hbm, idx_v, rows_v, gsem, ssem):
        wid = lax.axis_index("s") * _NUM_CORES + lax.axis_index("c")
        base = wid * b_per_w
        pltpu.sync_copy(idx_hbm.at[pl.ds(base, b_per_w)], idx_v)

        def start_gather(g, buf):
            return pltpu.async_copy(
                table_hbm.at[idx_v.at[pl.ds(g * _CHUNK, _CHUNK)]],
                rows_v.at[buf],
                gsem,
            )

        def start_store(g, buf):
            return pltpu.async_copy(
                rows_v.at[buf],
                out_hbm.at[pl.ds(base + g * _CHUNK, _CHUNK)],
                ssem,
            )

        gathers = [start_gather(0, 0)]
        stores = []
        for g in range(nchunks):
            buf = g & 1
            if g + 1 < nchunks:
                if stores:
                    stores.pop(0).wait()
                gathers.append(start_gather(g + 1, (g + 1) & 1))
            gathers.pop(0).wait()
            stores.append(start_store(g, buf))
        for s in stores:
            s.wait()

    return gather


def kernel(indices, weight):
    flat = indices.reshape(-1)
    gather = _make_gather(weight.shape[0], weight.shape[1], flat.shape[0])
    out = gather(weight, flat)
    return out.reshape(indices.shape + (weight.shape[-1],))
